# baseline (speedup 1.0000x reference)
import jax
import jax.numpy as jnp
from jax import lax
from jax.experimental import pallas as pl
from jax.experimental.pallas import tpu as pltpu
from jax.experimental.pallas import tpu_sc as plsc

HIDDEN = 64
SEQ = 200
BATCH = 4096
NUM_WORKERS = 32
BPW = BATCH // NUM_WORKERS
HT = HIDDEN // 8
PPS = 2
NSTEP = SEQ // PPS
PITCH = 129
G_BYTES = PPS * BPW * HIDDEN * 4
O_BYTES = PPS * HT * 8 * 128 * 4


def _splat(x):
    return jnp.full((16,), x, dtype=jnp.int32)


def _combine(rows, stg, pos_v, s):
    iota = lax.iota(jnp.int32, 16)
    th_c = [jax.lax.shift_right_logical(iota + c * 16, 3)
            for c in range(HIDDEN // 16)]
    hl_c = [jax.lax.bitwise_and(iota + c * 16, 7)
            for c in range(HIDDEN // 16)]
    pos_c = [pos_v[s, pl.ds(c * 16, 16)] for c in range(HIDDEN // 16)]

    def b_body(b, carry):
        b_vec = _splat(b)
        for c in range(HIDDEN // 16):
            v = jnp.maximum(rows[b, pl.ds(c * 16, 16)] + pos_c[c], 0.0)
            plsc.store_scatter(stg, [th_c[c], hl_c[c], b_vec], v)
        return carry

    lax.fori_loop(0, BPW, b_body, 0, unroll=8)


def _sc_body(idx_hbm, wtab_hbm, ptab_hbm, out_hbm,
             idx_all, rows, stg, pos_v, semg0, semg1, semo0, semo1):
    nc = 2
    wid = lax.axis_index("s") * nc + lax.axis_index("c")
    semg = (semg0, semg1)
    semo = (semo0, semo1)

    pltpu.sync_copy(ptab_hbm, pos_v)
    for p2 in range(PPS):
        pltpu.sync_copy(idx_hbm.at[:, p2, pl.ds(wid * BPW, BPW)],
                        idx_all.at[:, 0, pl.ds(p2 * BPW, BPW)])

    def stg_out_view(slot):
        return stg.at[slot].at[:, :, :, pl.ds(0, 128)]

    def start_gather(slot, sem, g):
        pltpu.async_copy(wtab_hbm.at[idx_all.at[g, 0]],
                         rows.at[slot].at[0], sem)

    start_gather(0, semg[0], 0)

    def pair_body(t, carry):
        g0 = 2 * t
        for par in range(2):
            g = g0 + par
            nxt = g + 1
            nxt = jnp.where(nxt >= NSTEP, 0, nxt)
            start_gather(1 - par, semg[1 - par], nxt)
            pltpu.make_async_copy(wtab_hbm.at[idx_all.at[0, 0]],
                                  rows.at[par].at[0], semg[par]).wait()
            @pl.when(g >= 2)
            def _():
                pltpu.make_async_copy(out_hbm.at[pl.ds(0, PPS), :, 0],
                                      stg_out_view(par), semo[par]).wait()
            for p2 in range(PPS):
                _combine(rows.at[par].at[0].at[pl.ds(p2 * BPW, BPW)],
                         stg.at[par].at[p2], pos_v, PPS * g + p2)
            pltpu.async_copy(stg_out_view(par),
                             out_hbm.at[pl.ds(PPS * g, PPS), :, wid],
                             semo[par])
        return carry

    lax.fori_loop(0, NSTEP // 2, pair_body, 0)

    pltpu.make_async_copy(wtab_hbm.at[idx_all.at[0, 0]], rows.at[0].at[0],
                          semg[0]).wait()
    for par in range(2):
        pltpu.make_async_copy(out_hbm.at[pl.ds(0, PPS), :, 0],
                              stg_out_view(par), semo[par]).wait()


@jax.jit
def kernel(input_seq, word_table, pos_table):
    batch, seq = input_seq.shape
    idx_t = jnp.swapaxes(input_seq, 0, 1).astype(jnp.int32)
    idx_t = idx_t.reshape(NSTEP, PPS, batch)

    mesh = plsc.VectorSubcoreMesh(core_axis_name="c", subcore_axis_name="s")
    run = pl.kernel(
        _sc_body,
        out_type=jax.ShapeDtypeStruct((SEQ, HT, NUM_WORKERS, 8, 128),
                                      jnp.float32),
        mesh=mesh,
        scratch_types=(
            [pltpu.VMEM((NSTEP, 1, PPS * BPW), jnp.int32),
             pltpu.VMEM((2, 1, PPS * BPW, HIDDEN), jnp.float32),
             pltpu.VMEM((2, PPS, HT, 8, PITCH), jnp.float32),
             pltpu.VMEM((SEQ, HIDDEN), jnp.float32)]
            + [pltpu.SemaphoreType.DMA] * 4
        ),
        compiler_params=pltpu.CompilerParams(use_tc_tiling_on_sc=False,
                                             needs_layout_passes=False),
    )
    out5d = run(idx_t, word_table, pos_table)
    return out5d.transpose(2, 4, 0, 1, 3).reshape(batch, seq, HIDDEN)


# speedup vs baseline: 1.0217x; 1.0217x over previous
import jax
import jax.numpy as jnp
from jax import lax
from jax.experimental import pallas as pl
from jax.experimental.pallas import tpu as pltpu
from jax.experimental.pallas import tpu_sc as plsc

HIDDEN = 64
SEQ = 200
BATCH = 4096
NUM_WORKERS = 32
BPW = BATCH // NUM_WORKERS
HT = HIDDEN // 8
DEPTH = 4
AHEAD = 3
PITCH = 129


def _splat(x):
    return jnp.full((16,), x, dtype=jnp.int32)


def _combine(rows, stg, pos_v, s):
    iota = lax.iota(jnp.int32, 16)
    th_c = [jax.lax.shift_right_logical(iota + c * 16, 3)
            for c in range(HIDDEN // 16)]
    hl_c = [jax.lax.bitwise_and(iota + c * 16, 7)
            for c in range(HIDDEN // 16)]
    pos_c = [pos_v[s, pl.ds(c * 16, 16)] for c in range(HIDDEN // 16)]

    def b_body(b, carry):
        b_vec = _splat(b)
        for c in range(HIDDEN // 16):
            v = jnp.maximum(rows[b, pl.ds(c * 16, 16)] + pos_c[c], 0.0)
            plsc.store_scatter(stg, [th_c[c], hl_c[c], b_vec], v)
        return carry

    lax.fori_loop(0, BPW, b_body, 0, unroll=4)


def _sc_body(idx_hbm, wtab_hbm, ptab_hbm, out_hbm,
             idx_all, rows, stg, pos_v, *sems):
    nc = 2
    wid = lax.axis_index("s") * nc + lax.axis_index("c")
    semg = sems[:DEPTH]
    semo = sems[DEPTH:]

    pltpu.sync_copy(ptab_hbm, pos_v)
    pltpu.sync_copy(idx_hbm.at[:, pl.ds(wid * BPW, BPW)], idx_all)

    def stg_out_view(slot):
        return stg.at[slot].at[:, :, pl.ds(0, 128)]

    def start_gather(slot, sem, s):
        pltpu.async_copy(wtab_hbm.at[idx_all.at[s]], rows.at[slot], sem)

    def drain_gather(slot, sem):
        pltpu.make_async_copy(wtab_hbm.at[pl.ds(0, BPW)], rows.at[slot],
                              sem).wait()

    def drain_out(slot, sem):
        pltpu.make_async_copy(out_hbm.at[0, :, 0], stg_out_view(slot),
                              sem).wait()

    for p in range(AHEAD):
        start_gather(p, semg[p], p)

    def quad_body(t, carry):
        s0 = DEPTH * t
        for par in range(DEPTH):
            s = s0 + par
            nxt = s + AHEAD
            nxt = jnp.where(nxt >= SEQ, nxt - SEQ, nxt)
            nslot = (par + AHEAD) % DEPTH
            start_gather(nslot, semg[nslot], nxt)
            drain_gather(par, semg[par])
            @pl.when(s >= DEPTH)
            def _():
                drain_out(par, semo[par])
            _combine(rows.at[par], stg.at[par], pos_v, s)
            pltpu.async_copy(stg_out_view(par), out_hbm.at[s, :, wid],
                             semo[par])
        return carry

    lax.fori_loop(0, SEQ // DEPTH, quad_body, 0)

    for p in range(AHEAD):
        drain_gather(p, semg[p])
    for p in range(DEPTH):
        drain_out(p, semo[p])


@jax.jit
def kernel(input_seq, word_table, pos_table):
    batch, seq = input_seq.shape
    idx_t = jnp.swapaxes(input_seq, 0, 1).astype(jnp.int32)

    mesh = plsc.VectorSubcoreMesh(core_axis_name="c", subcore_axis_name="s")
    run = pl.kernel(
        _sc_body,
        out_type=jax.ShapeDtypeStruct((SEQ, HT, NUM_WORKERS, 8, 128),
                                      jnp.float32),
        mesh=mesh,
        scratch_types=(
            [pltpu.VMEM((SEQ, BPW), jnp.int32),
             pltpu.VMEM((DEPTH, BPW, HIDDEN), jnp.float32),
             pltpu.VMEM((DEPTH, HT, 8, PITCH), jnp.float32),
             pltpu.VMEM((SEQ, HIDDEN), jnp.float32)]
            + [pltpu.SemaphoreType.DMA] * (2 * DEPTH)
        ),
        compiler_params=pltpu.CompilerParams(use_tc_tiling_on_sc=False,
                                             needs_layout_passes=False),
    )
    out5d = run(idx_t, word_table, pos_table)
    return out5d.transpose(2, 4, 0, 1, 3).reshape(batch, seq, HIDDEN)


# speedup vs baseline: 1.0238x; 1.0021x over previous
import jax
import jax.numpy as jnp
from jax import lax
from jax.experimental import pallas as pl
from jax.experimental.pallas import tpu as pltpu
from jax.experimental.pallas import tpu_sc as plsc

HIDDEN = 64
SEQ = 200
BATCH = 4096
NUM_WORKERS = 32
BPW = BATCH // NUM_WORKERS
HT = HIDDEN // 8
DEPTH = 4
AHEAD = 3
PITCH = 129


def _splat(x):
    return jnp.full((16,), x, dtype=jnp.int32)


def _combine(rows, stg, pos_v, s):
    iota = lax.iota(jnp.int32, 16)
    th_c = [jax.lax.shift_right_logical(iota + c * 16, 3)
            for c in range(HIDDEN // 16)]
    hl_c = [jax.lax.bitwise_and(iota + c * 16, 7)
            for c in range(HIDDEN // 16)]
    pos_c = [pos_v[s, pl.ds(c * 16, 16)] for c in range(HIDDEN // 16)]

    @pl.loop(0, BPW, unroll=4)
    def b_body(b):
        b_vec = _splat(b)
        for c in range(HIDDEN // 16):
            v = jnp.maximum(rows[b, pl.ds(c * 16, 16)] + pos_c[c], 0.0)
            plsc.store_scatter(stg, [th_c[c], hl_c[c], b_vec], v)


def _sc_body(idx_hbm, wtab_hbm, ptab_hbm, out_hbm,
             idx_all, rows, stg, pos_v, *sems):
    nc = 2
    wid = lax.axis_index("s") * nc + lax.axis_index("c")
    semg = sems[:DEPTH]
    semo = sems[DEPTH:]

    pltpu.sync_copy(ptab_hbm, pos_v)
    pltpu.sync_copy(idx_hbm.at[:, pl.ds(wid * BPW, BPW)], idx_all)

    def stg_out_view(slot):
        return stg.at[slot].at[:, :, pl.ds(0, 128)]

    def start_gather(slot, sem, s):
        pltpu.async_copy(wtab_hbm.at[idx_all.at[s]], rows.at[slot], sem)

    def drain_gather(slot, sem):
        pltpu.make_async_copy(wtab_hbm.at[pl.ds(0, BPW)], rows.at[slot],
                              sem).wait()

    def drain_out(slot, sem):
        pltpu.make_async_copy(out_hbm.at[0, :, 0], stg_out_view(slot),
                              sem).wait()

    for p in range(AHEAD):
        start_gather(p, semg[p], p)

    def quad_body(t, carry):
        s0 = DEPTH * t
        for par in range(DEPTH):
            s = s0 + par
            nxt = s + AHEAD
            nxt = jnp.where(nxt >= SEQ, nxt - SEQ, nxt)
            nslot = (par + AHEAD) % DEPTH
            start_gather(nslot, semg[nslot], nxt)
            drain_gather(par, semg[par])
            @pl.when(s >= DEPTH)
            def _():
                drain_out(par, semo[par])
            _combine(rows.at[par], stg.at[par], pos_v, s)
            pltpu.async_copy(stg_out_view(par), out_hbm.at[s, :, wid],
                             semo[par])
        return carry

    lax.fori_loop(0, SEQ // DEPTH, quad_body, 0)

    for p in range(AHEAD):
        drain_gather(p, semg[p])
    for p in range(DEPTH):
        drain_out(p, semo[p])


@jax.jit
def kernel(input_seq, word_table, pos_table):
    batch, seq = input_seq.shape
    idx_t = jnp.swapaxes(input_seq, 0, 1).astype(jnp.int32)

    mesh = plsc.VectorSubcoreMesh(core_axis_name="c", subcore_axis_name="s")
    run = pl.kernel(
        _sc_body,
        out_type=jax.ShapeDtypeStruct((SEQ, HT, NUM_WORKERS, 8, 128),
                                      jnp.float32),
        mesh=mesh,
        scratch_types=(
            [pltpu.VMEM((SEQ, BPW), jnp.int32),
             pltpu.VMEM((DEPTH, BPW, HIDDEN), jnp.float32),
             pltpu.VMEM((DEPTH, HT, 8, PITCH), jnp.float32),
             pltpu.VMEM((SEQ, HIDDEN), jnp.float32)]
            + [pltpu.SemaphoreType.DMA] * (2 * DEPTH)
        ),
        compiler_params=pltpu.CompilerParams(use_tc_tiling_on_sc=False,
                                             needs_layout_passes=False),
    )
    out5d = run(idx_t, word_table, pos_table)
    return out5d.transpose(2, 4, 0, 1, 3).reshape(batch, seq, HIDDEN)


# speedup vs baseline: 1.0249x; 1.0011x over previous
import jax
import jax.numpy as jnp
from jax import lax
from jax.experimental import pallas as pl
from jax.experimental.pallas import tpu as pltpu
from jax.experimental.pallas import tpu_sc as plsc

HIDDEN = 64
SEQ = 200
BATCH = 4096
NUM_WORKERS = 32
BPW = BATCH // NUM_WORKERS
HT = HIDDEN // 8
DEPTH = 4
AHEAD = 3
PITCH = 129


def _splat(x):
    return jnp.full((16,), x, dtype=jnp.int32)


def _combine(rows, stg, pos_v, s):
    iota = lax.iota(jnp.int32, 16)
    th_c = [jax.lax.shift_right_logical(iota + c * 16, 3)
            for c in range(HIDDEN // 16)]
    hl_c = [jax.lax.bitwise_and(iota + c * 16, 7)
            for c in range(HIDDEN // 16)]
    pos_c = [pos_v[s, pl.ds(c * 16, 16)] for c in range(HIDDEN // 16)]

    def b_body(b, carry):
        b_vec = _splat(b)
        for c in range(HIDDEN // 16):
            v = jnp.maximum(rows[b, pl.ds(c * 16, 16)] + pos_c[c], 0.0)
            plsc.store_scatter(stg, [th_c[c], hl_c[c], b_vec], v)
        return carry

    lax.fori_loop(0, BPW, b_body, 0, unroll=4)


def _sc_body(idx_hbm, wtab_hbm, ptab_hbm, out_hbm,
             idx_all, rows, stg, pos_v, *sems):
    nc = 2
    wid = lax.axis_index("s") * nc + lax.axis_index("c")
    semg = sems[:DEPTH]
    semo = sems[DEPTH:]

    pltpu.sync_copy(ptab_hbm, pos_v)
    pltpu.sync_copy(idx_hbm.at[:, pl.ds(wid * BPW, BPW)], idx_all)

    def stg_out_view(slot):
        return stg.at[slot].at[:, :, pl.ds(0, 128)]

    def start_gather(slot, sem, s):
        pltpu.async_copy(wtab_hbm.at[idx_all.at[s]], rows.at[slot], sem)

    def drain_gather(slot, sem):
        pltpu.make_async_copy(wtab_hbm.at[pl.ds(0, BPW)], rows.at[slot],
                              sem).wait()

    def drain_out(slot, sem):
        pltpu.make_async_copy(out_hbm.at[0, :, 0], stg_out_view(slot),
                              sem).wait()

    for p in range(AHEAD):
        start_gather(p, semg[p], p)

    def quad_body(t, carry):
        s0 = DEPTH * t
        for par in range(DEPTH):
            s = s0 + par
            nxt = s + AHEAD
            nxt = jnp.where(nxt >= SEQ, nxt - SEQ, nxt)
            nslot = (par + AHEAD) % DEPTH
            start_gather(nslot, semg[nslot], nxt)
            drain_gather(par, semg[par])
            @pl.when(s >= DEPTH)
            def _():
                drain_out(par, semo[par])
            _combine(rows.at[par], stg.at[par], pos_v, s)
            pltpu.async_copy(stg_out_view(par), out_hbm.at[s, :, wid],
                             semo[par])
        return carry

    lax.fori_loop(0, SEQ // DEPTH, quad_body, 0)

    for p in range(AHEAD):
        drain_gather(p, semg[p])
    for p in range(DEPTH):
        drain_out(p, semo[p])


@jax.jit
def kernel(input_seq, word_table, pos_table):
    batch, seq = input_seq.shape
    idx_t = jnp.swapaxes(input_seq, 0, 1).astype(jnp.int32)

    mesh = plsc.VectorSubcoreMesh(core_axis_name="c", subcore_axis_name="s")
    run = pl.kernel(
        _sc_body,
        out_type=jax.ShapeDtypeStruct((SEQ, HT, NUM_WORKERS, 8, 128),
                                      jnp.float32),
        mesh=mesh,
        scratch_types=(
            [pltpu.VMEM((SEQ, BPW), jnp.int32),
             pltpu.VMEM((DEPTH, BPW, HIDDEN), jnp.float32),
             pltpu.VMEM((DEPTH, HT, 8, PITCH), jnp.float32),
             pltpu.VMEM((SEQ, HIDDEN), jnp.float32)]
            + [pltpu.SemaphoreType.DMA] * (2 * DEPTH)
        ),
        compiler_params=pltpu.CompilerParams(use_tc_tiling_on_sc=False,
                                             needs_layout_passes=False),
    )
    out5d = run(idx_t, word_table, pos_table)
    return out5d.transpose(2, 4, 0, 1, 3).reshape(batch, seq, HIDDEN)
